# baseline (speedup 1.0000x reference)
import functools

import jax
import jax.numpy as jnp
from jax.experimental import pallas as pl
from jax.experimental.pallas import tpu as pltpu


def _gelu(x):
    return 0.5 * x * (1.0 + jax.lax.erf(x * 0.7071067811865476))


def _dot_t(x, w):
    return jax.lax.dot_general(
        x, w, dimension_numbers=(((1,), (1,)), ((), ())),
        preferred_element_type=jnp.float32)


def _fused_kernel(fm_ref, ft_ref, fi_ref, ri_ref,
                  l1w_ref, l1b_ref, l2w_ref, l2b_ref, temb_ref,
                  w1_ref, b1_ref, w2_ref, b2_ref,
                  out_ref, mw_ref):
    x = fm_ref[...].astype(jnp.bfloat16)
    h = _gelu(_dot_t(x, w1_ref[...]) + b1_ref[...])
    fm = _dot_t(h.astype(jnp.bfloat16), w2_ref[...]) + b2_ref[...]

    r = _gelu(_dot_t(ri_ref[...], l1w_ref[...]) + l1b_ref[...]) + temb_ref[...]
    logits = _dot_t(r, l2w_ref[...]) + l2b_ref[...]

    mx = jnp.max(logits, axis=1, keepdims=True)
    mn = jnp.min(logits, axis=1, keepdims=True)
    idx = jax.lax.broadcasted_iota(jnp.int32, logits.shape, 1)
    drop_idx = jnp.max(jnp.where(logits == mn, idx, -1), axis=1, keepdims=True)
    e = jnp.where(idx != drop_idx, jnp.exp(logits - mx), 0.0)
    mw = e / jnp.sum(e, axis=1, keepdims=True)
    mw_ref[...] = mw

    out_ref[...] = (ft_ref[...] * mw[:, 0:1]
                    + fm * mw[:, 1:2]
                    + fi_ref[...] * mw[:, 2:3])


@jax.jit
def kernel(F_M, F_T, F_I, router_input, l1_w, l1_b, l2_w, l2_b, task_emb,
           mlp_w1, mlp_b1, mlp_w2, mlp_b2):
    B, T, D = F_T.shape
    D2 = F_M.shape[-1]
    E = l1_w.shape[0]
    N = B * T
    TK = 512

    fm2 = F_M.reshape(N, D2)
    ft2 = F_T.reshape(N, D)
    fi2 = F_I.reshape(N, D)
    ri2 = router_input.reshape(N, D2)
    temb = task_emb[1].reshape(1, E)

    grid = (N // TK,)

    def tok_spec(width):
        return pl.BlockSpec((TK, width), lambda i: (i, 0))

    def const_spec(shape):
        return pl.BlockSpec(shape, lambda i: (0,) * len(shape))

    out, mw = pl.pallas_call(
        _fused_kernel,
        grid=grid,
        in_specs=[
            tok_spec(D2),
            tok_spec(D),
            tok_spec(D),
            tok_spec(D2),
            const_spec((E, D2)),
            const_spec((1, E)),
            const_spec((3, E)),
            const_spec((1, 3)),
            const_spec((1, E)),
            const_spec((D2, D2)),
            const_spec((1, D2)),
            const_spec((D, D2)),
            const_spec((1, D)),
        ],
        out_specs=[tok_spec(D), tok_spec(3)],
        out_shape=[
            jax.ShapeDtypeStruct((N, D), jnp.float32),
            jax.ShapeDtypeStruct((N, 3), jnp.float32),
        ],
    )(fm2, ft2, fi2, ri2,
      l1_w, l1_b.reshape(1, E), l2_w, l2_b.reshape(1, 3), temb,
      mlp_w1.astype(jnp.bfloat16), mlp_b1.reshape(1, D2),
      mlp_w2.astype(jnp.bfloat16), mlp_b2.reshape(1, D))

    return out.reshape(B, T, D), mw.reshape(B, T, 3)


# speedup vs baseline: 1.0765x; 1.0765x over previous
import functools

import jax
import jax.numpy as jnp
from jax.experimental import pallas as pl
from jax.experimental.pallas import tpu as pltpu


def _gelu(x):
    return 0.5 * x * (1.0 + jax.lax.erf(x * 0.7071067811865476))


def _dot_t(x, w):
    return jax.lax.dot_general(
        x, w, dimension_numbers=(((1,), (1,)), ((), ())),
        preferred_element_type=jnp.float32)


_N_SUB = 2


def _fused_kernel(fm_ref, ft_ref, fi_ref, ri_ref,
                  l1w_ref, l1b_ref, l2w_ref, l2b_ref, temb_ref,
                  w1_ref, b1_ref, w2_ref, b2_ref,
                  out_ref, mw_ref):
    sub = out_ref.shape[0] // _N_SUB
    for k in range(_N_SUB):
        s = pl.ds(k * sub, sub)
        h = _gelu(_dot_t(fm_ref[s, :], w1_ref[...]) + b1_ref[...])
        fm = _dot_t(h, w2_ref[...]) + b2_ref[...]

        r = (_gelu(_dot_t(ri_ref[s, :], l1w_ref[...]) + l1b_ref[...])
             + temb_ref[...])
        logits = _dot_t(r, l2w_ref[...]) + l2b_ref[...]

        mx = jnp.max(logits, axis=1, keepdims=True)
        mn = jnp.min(logits, axis=1, keepdims=True)
        idx = jax.lax.broadcasted_iota(jnp.int32, logits.shape, 1)
        drop_idx = jnp.max(jnp.where(logits == mn, idx, -1),
                           axis=1, keepdims=True)
        e = jnp.where(idx != drop_idx, jnp.exp(logits - mx), 0.0)
        mw = e / jnp.sum(e, axis=1, keepdims=True)
        mw_ref[s, :] = mw

        out_ref[s, :] = (ft_ref[s, :] * mw[:, 0:1]
                         + fm * mw[:, 1:2]
                         + fi_ref[s, :] * mw[:, 2:3])


@jax.jit
def kernel(F_M, F_T, F_I, router_input, l1_w, l1_b, l2_w, l2_b, task_emb,
           mlp_w1, mlp_b1, mlp_w2, mlp_b2):
    B, T, D = F_T.shape
    D2 = F_M.shape[-1]
    E = l1_w.shape[0]
    N = B * T
    TK = 512

    fm2 = F_M.reshape(N, D2)
    ft2 = F_T.reshape(N, D)
    fi2 = F_I.reshape(N, D)
    ri2 = router_input.reshape(N, D2)
    temb = task_emb[1].reshape(1, E)

    grid = (N // TK,)

    def tok_spec(width):
        return pl.BlockSpec((TK, width), lambda i: (i, 0))

    def const_spec(shape):
        return pl.BlockSpec(shape, lambda i: (0,) * len(shape))

    out, mw = pl.pallas_call(
        _fused_kernel,
        grid=grid,
        in_specs=[
            tok_spec(D2),
            tok_spec(D),
            tok_spec(D),
            tok_spec(D2),
            const_spec((E, D2)),
            const_spec((1, E)),
            const_spec((3, E)),
            const_spec((1, 3)),
            const_spec((1, E)),
            const_spec((D2, D2)),
            const_spec((1, D2)),
            const_spec((D, D2)),
            const_spec((1, D)),
        ],
        out_specs=[tok_spec(D), tok_spec(3)],
        out_shape=[
            jax.ShapeDtypeStruct((N, D), jnp.float32),
            jax.ShapeDtypeStruct((N, 3), jnp.float32),
        ],
    )(fm2, ft2, fi2, ri2,
      l1_w, l1_b.reshape(1, E), l2_w, l2_b.reshape(1, 3), temb,
      mlp_w1, mlp_b1.reshape(1, D2), mlp_w2, mlp_b2.reshape(1, D))

    return out.reshape(B, T, D), mw.reshape(B, T, 3)


# speedup vs baseline: 1.0887x; 1.0113x over previous
import functools

import jax
import jax.numpy as jnp
from jax.experimental import pallas as pl
from jax.experimental.pallas import tpu as pltpu


def _gelu(x):
    return 0.5 * x * (1.0 + jax.lax.erf(x * 0.7071067811865476))


def _dot_t(x, w):
    return jax.lax.dot_general(
        x, w, dimension_numbers=(((1,), (1,)), ((), ())),
        preferred_element_type=jnp.float32)


_N_SUB = 2


def _fused_kernel(fm_ref, ft_ref, fi_ref, ri_ref,
                  l1w_ref, l1b_ref, l2w_ref, l2b_ref, temb_ref,
                  w1_ref, b1_ref, w2_ref, b2_ref,
                  out_ref, mw_ref, w1b_ref, w2b_ref):
    @pl.when(pl.program_id(0) == 0)
    def _cast_weights():
        w1b_ref[...] = w1_ref[...].astype(jnp.bfloat16)
        w2b_ref[...] = w2_ref[...].astype(jnp.bfloat16)

    sub = out_ref.shape[0] // _N_SUB
    for k in range(_N_SUB):
        s = pl.ds(k * sub, sub)
        x = fm_ref[s, :].astype(jnp.bfloat16)
        h = _gelu(_dot_t(x, w1b_ref[...]) + b1_ref[...])
        fm = _dot_t(h.astype(jnp.bfloat16), w2b_ref[...]) + b2_ref[...]

        r = (_gelu(_dot_t(ri_ref[s, :], l1w_ref[...]) + l1b_ref[...])
             + temb_ref[...])
        logits = _dot_t(r, l2w_ref[...]) + l2b_ref[...]

        mx = jnp.max(logits, axis=1, keepdims=True)
        mn = jnp.min(logits, axis=1, keepdims=True)
        idx = jax.lax.broadcasted_iota(jnp.int32, logits.shape, 1)
        drop_idx = jnp.max(jnp.where(logits == mn, idx, -1),
                           axis=1, keepdims=True)
        e = jnp.where(idx != drop_idx, jnp.exp(logits - mx), 0.0)
        mw = e / jnp.sum(e, axis=1, keepdims=True)
        mw_ref[s, :] = mw

        out_ref[s, :] = (ft_ref[s, :] * mw[:, 0:1]
                         + fm * mw[:, 1:2]
                         + fi_ref[s, :] * mw[:, 2:3])


@jax.jit
def kernel(F_M, F_T, F_I, router_input, l1_w, l1_b, l2_w, l2_b, task_emb,
           mlp_w1, mlp_b1, mlp_w2, mlp_b2):
    B, T, D = F_T.shape
    D2 = F_M.shape[-1]
    E = l1_w.shape[0]
    N = B * T
    TK = 512

    fm2 = F_M.reshape(N, D2)
    ft2 = F_T.reshape(N, D)
    fi2 = F_I.reshape(N, D)
    ri2 = router_input.reshape(N, D2)
    temb = task_emb[1].reshape(1, E)

    grid = (N // TK,)

    def tok_spec(width):
        return pl.BlockSpec((TK, width), lambda i: (i, 0))

    def const_spec(shape):
        return pl.BlockSpec(shape, lambda i: (0,) * len(shape))

    out, mw = pl.pallas_call(
        _fused_kernel,
        grid=grid,
        in_specs=[
            tok_spec(D2),
            tok_spec(D),
            tok_spec(D),
            tok_spec(D2),
            const_spec((E, D2)),
            const_spec((1, E)),
            const_spec((3, E)),
            const_spec((1, 3)),
            const_spec((1, E)),
            const_spec((D2, D2)),
            const_spec((1, D2)),
            const_spec((D, D2)),
            const_spec((1, D)),
        ],
        out_specs=[tok_spec(D), tok_spec(3)],
        out_shape=[
            jax.ShapeDtypeStruct((N, D), jnp.float32),
            jax.ShapeDtypeStruct((N, 3), jnp.float32),
        ],
        scratch_shapes=[
            pltpu.VMEM((D2, D2), jnp.bfloat16),
            pltpu.VMEM((D, D2), jnp.bfloat16),
        ],
    )(fm2, ft2, fi2, ri2,
      l1_w, l1_b.reshape(1, E), l2_w, l2_b.reshape(1, 3), temb,
      mlp_w1, mlp_b1.reshape(1, D2), mlp_w2, mlp_b2.reshape(1, D))

    return out.reshape(B, T, D), mw.reshape(B, T, 3)
